# Initial kernel scaffold; baseline (speedup 1.0000x reference)
import jax
import jax.numpy as jnp
from jax.experimental import pallas as pl


def kernel(X_sparse, X_dense, tables, W_cate, b_cate, W_nume, b_nume):
    raise NotImplementedError("write your pallas kernel here")


# speedup vs baseline: 2.5682x; 2.5682x over previous
import jax
import jax.numpy as jnp
from jax import lax
from jax.experimental import pallas as pl
from jax.experimental.pallas import tpu as pltpu, tpu_sc as plsc

_NUM_FIELDS = 26
_VOCAB = 1000000
_BATCH = 16384
_DENSE = 13

_NW = 32
_RPW = _BATCH // _NW
_CHUNK = 128
_NG = _NUM_FIELDS * _RPW // _CHUNK
_NCH = _RPW // 16
_FIRE = 8


def _body(table_hbm, idx_hbm, xd_hbm, wv_hbm, wnv_hbm, bias_hbm, out_hbm,
          idx_v, vals_v, xd_v, wv_v, wnv_v, bias_v, out_v, sem):
    wid = lax.axis_index("s") * 2 + lax.axis_index("c")
    base = wid * _RPW

    pltpu.sync_copy(idx_hbm.at[wid], idx_v)
    pltpu.sync_copy(xd_hbm.at[wid], xd_v)
    pltpu.sync_copy(wv_hbm, wv_v)
    pltpu.sync_copy(wnv_hbm, wnv_v)
    pltpu.sync_copy(bias_hbm, bias_v)

    def fire(g, _):
        pltpu.async_copy(table_hbm.at[idx_v.at[g]], vals_v.at[g], sem)

        @pl.when(g >= _FIRE)
        def _drain():
            pltpu.make_async_copy(table_hbm.at[idx_v.at[0]], vals_v.at[0],
                                  sem).wait()
        return 0

    lax.fori_loop(0, _NG, fire, 0, unroll=False)
    for _ in range(_FIRE):
        pltpu.make_async_copy(table_hbm.at[idx_v.at[0]], vals_v.at[0],
                              sem).wait()

    for c in range(_NCH):
        sub = c // 8
        col = (c % 8) * 16

        def fbody(f, acc):
            v = vals_v[4 * f + sub, pl.ds(col, 16)]
            return acc + wv_v[f, :] * v

        acc = lax.fori_loop(0, _NUM_FIELDS, fbody,
                            bias_v[0, :], unroll=False)

        def dbody(d, acc):
            x = xd_v[d, pl.ds(c * 16, 16)]
            return acc + wnv_v[d, :] * x

        acc = lax.fori_loop(0, _DENSE, dbody, acc, unroll=False)
        out_v[pl.ds(c * 16, 16)] = 1.0 / (1.0 + jnp.exp(-acc))

    pltpu.sync_copy(out_v, out_hbm.at[pl.ds(base, _RPW)])


def kernel(X_sparse, X_dense, tables, W_cate, b_cate, W_nume, b_nume):
    flat_idx = X_sparse + (jnp.arange(_NUM_FIELDS, dtype=jnp.int32) * _VOCAB)
    idx_prep = (flat_idx.T.reshape(_NUM_FIELDS, _NW, _RPW)
                .transpose(1, 0, 2).reshape(_NW, _NG, _CHUNK))
    xd_prep = (X_dense.T.reshape(_DENSE, _NW, _RPW)
               .transpose(1, 0, 2))
    table_flat = tables.reshape(_NUM_FIELDS * _VOCAB)
    wv = jnp.broadcast_to(W_cate[0][:, None], (_NUM_FIELDS, 16))
    wnv = jnp.broadcast_to(W_nume[0][:, None], (_DENSE, 16))
    bias = jnp.broadcast_to((b_cate + b_nume)[:, None], (1, 16))

    mesh = plsc.VectorSubcoreMesh(core_axis_name="c", subcore_axis_name="s")
    out = pl.kernel(
        _body,
        out_type=jax.ShapeDtypeStruct((_BATCH,), jnp.float32),
        mesh=mesh,
        scratch_types=[
            pltpu.VMEM((_NG, _CHUNK), jnp.int32),
            pltpu.VMEM((_NG, _CHUNK), jnp.float32),
            pltpu.VMEM((_DENSE, _RPW), jnp.float32),
            pltpu.VMEM((_NUM_FIELDS, 16), jnp.float32),
            pltpu.VMEM((_DENSE, 16), jnp.float32),
            pltpu.VMEM((1, 16), jnp.float32),
            pltpu.VMEM((_RPW,), jnp.float32),
            pltpu.SemaphoreType.DMA,
        ],
    )(table_flat, idx_prep, xd_prep, wv, wnv, bias)
    return out.reshape(_BATCH, 1)


# speedup vs baseline: 4.5702x; 1.7795x over previous
import jax
import jax.numpy as jnp
from jax import lax
from jax.experimental import pallas as pl
from jax.experimental.pallas import tpu as pltpu, tpu_sc as plsc

_NUM_FIELDS = 26
_VOCAB = 1000000
_BATCH = 16384
_DENSE = 13

_NW = 32
_RPW = _BATCH // _NW
_CHUNK = 128
_NG = _NUM_FIELDS * _RPW // _CHUNK
_NCH = _RPW // 16
_FIRE = 8


def _body(table_hbm, idx_hbm, xd_hbm, wv_hbm, wnv_hbm, bias_hbm, out_hbm,
          idx_v, vals_v, xd_v, wv_v, wnv_v, bias_v, out_v, sem):
    wid = lax.axis_index("s") * 2 + lax.axis_index("c")
    base = wid * _RPW

    pltpu.sync_copy(idx_hbm.at[wid], idx_v)
    pltpu.sync_copy(xd_hbm.at[wid], xd_v)
    pltpu.sync_copy(wv_hbm, wv_v)
    pltpu.sync_copy(wnv_hbm, wnv_v)
    pltpu.sync_copy(bias_hbm, bias_v)

    def fire(g, _):
        pltpu.async_copy(table_hbm.at[idx_v.at[g]], vals_v.at[g], sem)

        @pl.when(g >= _FIRE)
        def _drain():
            pltpu.make_async_copy(table_hbm.at[idx_v.at[0]], vals_v.at[0],
                                  sem).wait()
        return 0

    lax.fori_loop(0, _NG, fire, 0, unroll=False)
    for _ in range(_FIRE):
        pltpu.make_async_copy(table_hbm.at[idx_v.at[0]], vals_v.at[0],
                              sem).wait()

    for c in range(_NCH):
        sub = c // 8
        col = (c % 8) * 16

        def fbody(f, acc):
            v = vals_v[4 * f + sub, pl.ds(col, 16)]
            return acc + wv_v[f, :] * v

        acc = lax.fori_loop(0, _NUM_FIELDS, fbody,
                            bias_v[0, :], unroll=False)

        def dbody(d, acc):
            x = xd_v[d, pl.ds(c * 16, 16)]
            return acc + wnv_v[d, :] * x

        acc = lax.fori_loop(0, _DENSE, dbody, acc, unroll=False)
        out_v[pl.ds(c * 16, 16)] = 1.0 / (1.0 + jnp.exp(-acc))

    pltpu.sync_copy(out_v, out_hbm.at[pl.ds(base, _RPW)])


def kernel(X_sparse, X_dense, tables, W_cate, b_cate, W_nume, b_nume):
    flat_idx = X_sparse + (jnp.arange(_NUM_FIELDS, dtype=jnp.int32) * _VOCAB)
    idx_prep = (flat_idx.T.reshape(_NUM_FIELDS, _NW, _RPW)
                .transpose(1, 0, 2).reshape(_NW, _NG, _CHUNK))
    xd_prep = (X_dense.T.reshape(_DENSE, _NW, _RPW)
               .transpose(1, 0, 2))
    table_2d = jnp.concatenate([tables[f, :, 0] for f in range(_NUM_FIELDS)])
    wv = jnp.broadcast_to(W_cate[0][:, None], (_NUM_FIELDS, 16))
    wnv = jnp.broadcast_to(W_nume[0][:, None], (_DENSE, 16))
    bias = jnp.broadcast_to((b_cate + b_nume)[:, None], (1, 16))

    mesh = plsc.VectorSubcoreMesh(core_axis_name="c", subcore_axis_name="s")
    out = pl.kernel(
        _body,
        out_type=jax.ShapeDtypeStruct((_BATCH,), jnp.float32),
        mesh=mesh,
        scratch_types=[
            pltpu.VMEM((_NG, _CHUNK), jnp.int32),
            pltpu.VMEM((_NG, _CHUNK), jnp.float32),
            pltpu.VMEM((_DENSE, _RPW), jnp.float32),
            pltpu.VMEM((_NUM_FIELDS, 16), jnp.float32),
            pltpu.VMEM((_DENSE, 16), jnp.float32),
            pltpu.VMEM((1, 16), jnp.float32),
            pltpu.VMEM((_RPW,), jnp.float32),
            pltpu.SemaphoreType.DMA,
        ],
    )(table_2d, idx_prep, xd_prep, wv, wnv, bias)
    return out.reshape(_BATCH, 1)


# speedup vs baseline: 40.8665x; 8.9420x over previous
import jax
import jax.numpy as jnp
from jax import lax
from jax.experimental import pallas as pl
from jax.experimental.pallas import tpu as pltpu, tpu_sc as plsc

_NUM_FIELDS = 26
_VOCAB = 1000000
_BATCH = 16384
_DENSE = 13

_PADV = 1001472
_NW = 32
_RPW = _BATCH // _NW
_CHUNK = 128
_NG = _NUM_FIELDS * _RPW // _CHUNK
_NCH = _RPW // 16
_FIRE = 8


def _body(table_hbm, idx_hbm, xd_hbm, wv_hbm, wnv_hbm, bias_hbm, out_hbm,
          idx_v, vals_v, xd_v, wv_v, wnv_v, bias_v, out_v, acc_v, sem):
    wid = lax.axis_index("s") * 2 + lax.axis_index("c")
    base = wid * _RPW

    pltpu.sync_copy(idx_hbm.at[wid], idx_v)
    pltpu.sync_copy(xd_hbm.at[wid], xd_v)
    pltpu.sync_copy(wv_hbm, wv_v)
    pltpu.sync_copy(wnv_hbm, wnv_v)
    pltpu.sync_copy(bias_hbm, bias_v)

    def fire(g, _):
        pltpu.async_copy(table_hbm.at[idx_v.at[g]], vals_v.at[g], sem)
        return 0

    lax.fori_loop(0, _NG, fire, 0, unroll=False)

    for c in range(_NCH):
        def dbody(d, acc):
            x = xd_v[d, pl.ds(c * 16, 16)]
            return acc + wnv_v[d, :] * x

        acc_v[pl.ds(c * 16, 16)] = lax.fori_loop(
            0, _DENSE, dbody, bias_v[0, :], unroll=False)

    def drain(f, _):
        for _j in range(4):
            pltpu.make_async_copy(table_hbm.at[idx_v.at[0]], vals_v.at[0],
                                  sem).wait()
        w = wv_v[f, :]
        for j in range(4):
            for k in range(8):
                o = j * 128 + k * 16
                acc_v[pl.ds(o, 16)] = (acc_v[pl.ds(o, 16)]
                                       + w * vals_v[4 * f + j,
                                                    pl.ds(k * 16, 16)])
        return 0

    lax.fori_loop(0, _NUM_FIELDS, drain, 0, unroll=False)

    for c in range(_NCH):
        acc = acc_v[pl.ds(c * 16, 16)]
        out_v[pl.ds(c * 16, 16)] = 1.0 / (1.0 + jnp.exp(-acc))

    pltpu.sync_copy(out_v, out_hbm.at[pl.ds(base, _RPW)])


def kernel(X_sparse, X_dense, tables, W_cate, b_cate, W_nume, b_nume):
    flat_idx = X_sparse + (jnp.arange(_NUM_FIELDS, dtype=jnp.int32) * _PADV)
    idx_prep = (flat_idx.T.reshape(_NUM_FIELDS, _NW, _RPW)
                .transpose(1, 0, 2).reshape(_NW, _NG, _CHUNK))
    xd_prep = (X_dense.T.reshape(_DENSE, _NW, _RPW)
               .transpose(1, 0, 2))
    table_2d = jnp.pad(tables, ((0, 0), (0, _PADV - _VOCAB), (0, 0))
                       ).reshape(_NUM_FIELDS * _PADV)
    wv = jnp.broadcast_to(W_cate[0][:, None], (_NUM_FIELDS, 16))
    wnv = jnp.broadcast_to(W_nume[0][:, None], (_DENSE, 16))
    bias = jnp.broadcast_to((b_cate + b_nume)[:, None], (1, 16))

    mesh = plsc.VectorSubcoreMesh(core_axis_name="c", subcore_axis_name="s")
    out = pl.kernel(
        _body,
        out_type=jax.ShapeDtypeStruct((_BATCH,), jnp.float32),
        mesh=mesh,
        scratch_types=[
            pltpu.VMEM((_NG, _CHUNK), jnp.int32),
            pltpu.VMEM((_NG, _CHUNK), jnp.float32),
            pltpu.VMEM((_DENSE, _RPW), jnp.float32),
            pltpu.VMEM((_NUM_FIELDS, 16), jnp.float32),
            pltpu.VMEM((_DENSE, 16), jnp.float32),
            pltpu.VMEM((1, 16), jnp.float32),
            pltpu.VMEM((_RPW,), jnp.float32),
            pltpu.VMEM((_RPW,), jnp.float32),
            pltpu.SemaphoreType.DMA,
        ],
    )(table_2d, idx_prep, xd_prep, wv, wnv, bias)
    return out.reshape(_BATCH, 1)
